# baseline (speedup 1.0000x reference)
import jax
import jax.numpy as jnp
from jax import lax
from jax.experimental import pallas as pl
from jax.experimental.pallas import tpu as pltpu
from jax.experimental.pallas import tpu_sc as plsc

_NUM_USERS = 1000000
_NUM_ITEMS = 1000000
_EMBED_DIM = 32
_TRAIN_INTERACTION_SIZE = 100000000
_WEIGHT_DECAY = 1e-4
_BATCH = 16384

_NC = 2
_NS = 16
_NW = _NC * _NS
_BPW = _BATCH // _NW
_G = _BPW // 16

_SPARSITY = _TRAIN_INTERACTION_SIZE / (_NUM_USERS * _NUM_ITEMS)
_BPR_SCALE = 1.0 / (_NUM_USERS * _SPARSITY * _BATCH)
_REG_SCALE = _WEIGHT_DECAY * 0.5 / _BATCH


def _sc_body(users_hbm, pos_hbm, neg_hbm, uflat_hbm, iflat_hbm, pop_hbm,
             diff_out, alpha_out, ssq_out,
             uidx_v, pidx_v, nidx_v, ut_v, pt_v, nt_v, alpha_v, diff_v,
             ssq_v,
             sem_u, sem_p, sem_n, sem_a):
    wid = lax.axis_index("s") * _NC + lax.axis_index("c")
    base = wid * _BPW

    pltpu.sync_copy(users_hbm.at[pl.ds(base, _BPW)], uidx_v)
    pltpu.sync_copy(pos_hbm.at[pl.ds(base, _BPW)], pidx_v)
    pltpu.sync_copy(neg_hbm.at[pl.ds(base, _BPW)], nidx_v)

    def issue(g, carry):
        iu_vec = uidx_v[pl.ds(g * 16, 16)]
        ip_vec = pidx_v[pl.ds(g * 16, 16)]
        iq_vec = nidx_v[pl.ds(g * 16, 16)]
        pltpu.async_copy(pop_hbm.at[iq_vec],
                         alpha_v.at[g // 8, pl.ds((g % 8) * 16, 16)], sem_a)
        for d in range(_EMBED_DIM):
            pltpu.async_copy(uflat_hbm.at[iu_vec + d * _NUM_USERS],
                             ut_v.at[d, pl.ds(g * 16, 16)], sem_u)
            pltpu.async_copy(iflat_hbm.at[ip_vec + d * _NUM_ITEMS],
                             pt_v.at[d, pl.ds(g * 16, 16)], sem_p)
            pltpu.async_copy(iflat_hbm.at[iq_vec + d * _NUM_ITEMS],
                             nt_v.at[d, pl.ds(g * 16, 16)], sem_n)
        return carry

    lax.fori_loop(0, _G, issue, 0)

    for d in range(_EMBED_DIM):
        pltpu.make_async_copy(diff_out.at[pl.ds(0, _BPW)],
                              ut_v.at[d], sem_u).wait()
        pltpu.make_async_copy(diff_out.at[pl.ds(0, _BPW)],
                              pt_v.at[d], sem_p).wait()
        pltpu.make_async_copy(diff_out.at[pl.ds(0, _BPW)],
                              nt_v.at[d], sem_n).wait()
    pltpu.make_async_copy(alpha_out.at[pl.ds(0, 4)], alpha_v, sem_a).wait()

    def compute(g, ssq):
        sl = pl.ds(g * 16, 16)
        pos_acc = jnp.zeros((16,), jnp.float32)
        neg_acc = jnp.zeros((16,), jnp.float32)
        for d in range(_EMBED_DIM):
            u = ut_v[d, sl]
            p = pt_v[d, sl]
            q = nt_v[d, sl]
            pos_acc = pos_acc + u * p
            neg_acc = neg_acc + u * q
            ssq = ssq + u * u + p * p + q * q
        diff_v[sl] = neg_acc - pos_acc
        return ssq

    ssq = lax.fori_loop(0, _G, compute, jnp.zeros((16,), jnp.float32))
    for r in range(16):
        ssq_v[pl.ds(r * 16, 16)] = ssq if r == 0 else jnp.zeros(
            (16,), jnp.float32)

    pltpu.sync_copy(diff_v, diff_out.at[pl.ds(base, _BPW)])
    pltpu.sync_copy(alpha_v, alpha_out.at[pl.ds(wid * 4, 4)])
    pltpu.sync_copy(ssq_v, ssq_out.at[wid])


_sc_gather = pl.kernel(
    _sc_body,
    out_type=[
        jax.ShapeDtypeStruct((_BATCH,), jnp.float32),
        jax.ShapeDtypeStruct((_BATCH // 128, 128), jnp.float32),
        jax.ShapeDtypeStruct((_NW, 256), jnp.float32),
    ],
    mesh=plsc.VectorSubcoreMesh(core_axis_name="c", subcore_axis_name="s"),
    scratch_types=[
        pltpu.VMEM((_BPW,), jnp.int32),
        pltpu.VMEM((_BPW,), jnp.int32),
        pltpu.VMEM((_BPW,), jnp.int32),
        pltpu.VMEM((_EMBED_DIM, _BPW), jnp.float32),
        pltpu.VMEM((_EMBED_DIM, _BPW), jnp.float32),
        pltpu.VMEM((_EMBED_DIM, _BPW), jnp.float32),
        pltpu.VMEM((4, 128), jnp.float32),
        pltpu.VMEM((_BPW,), jnp.float32),
        pltpu.VMEM((256,), jnp.float32),
        pltpu.SemaphoreType.DMA,
        pltpu.SemaphoreType.DMA,
        pltpu.SemaphoreType.DMA,
        pltpu.SemaphoreType.DMA,
    ],
)


def _loss_body(diff_ref, alpha_ref, ssq_ref, out_ref):
    x = diff_ref[...]
    a = alpha_ref[...]
    sp = jnp.maximum(x, 0.0) + jnp.log(1.0 + jnp.exp(-jnp.abs(x)))
    wb = jnp.sum(a * sp)
    lane = lax.broadcasted_iota(jnp.int32, (_NW, 256), 1)
    ssq = jnp.sum(jnp.where(lane < 16, ssq_ref[...], 0.0))
    out_ref[0, 0] = wb * _BPR_SCALE + ssq * _REG_SCALE


_loss_call = pl.pallas_call(
    _loss_body,
    out_shape=jax.ShapeDtypeStruct((1, 1), jnp.float32),
    in_specs=[
        pl.BlockSpec(memory_space=pltpu.VMEM),
        pl.BlockSpec(memory_space=pltpu.VMEM),
        pl.BlockSpec(memory_space=pltpu.VMEM),
    ],
    out_specs=pl.BlockSpec(memory_space=pltpu.SMEM),
)


def kernel(users, positive_items, negative_items, user_embedding,
           item_embedding, popularity):
    uflat = user_embedding.T.reshape(-1)
    iflat = item_embedding.T.reshape(-1)
    diff, alpha, ssq = _sc_gather(users, positive_items, negative_items,
                                  uflat, iflat, popularity)
    loss = _loss_call(diff.reshape(128, 128), alpha, ssq)
    return loss[0, 0]


# speedup vs baseline: 1.0012x; 1.0012x over previous
import jax
import jax.numpy as jnp
from jax import lax
from jax.experimental import pallas as pl
from jax.experimental.pallas import tpu as pltpu
from jax.experimental.pallas import tpu_sc as plsc

_NUM_USERS = 1000000
_NUM_ITEMS = 1000000
_EMBED_DIM = 32
_TRAIN_INTERACTION_SIZE = 100000000
_WEIGHT_DECAY = 1e-4
_BATCH = 16384

_NC = 2
_NS = 16
_NW = _NC * _NS
_BPW = _BATCH // _NW
_G = _BPW // 16

_SPARSITY = _TRAIN_INTERACTION_SIZE / (_NUM_USERS * _NUM_ITEMS)
_BPR_SCALE = 1.0 / (_NUM_USERS * _SPARSITY * _BATCH)
_REG_SCALE = _WEIGHT_DECAY * 0.5 / _BATCH


def _sc_body(uoff_hbm, poff_hbm, noff_hbm, neg_hbm,
             uflat_hbm, iflat_hbm, pop_hbm,
             diff_out, alpha_out, ssq_out,
             *scratch):
    uidx = scratch[0:_EMBED_DIM]
    pidx = scratch[_EMBED_DIM:2 * _EMBED_DIM]
    nidx = scratch[2 * _EMBED_DIM:3 * _EMBED_DIM]
    urow = scratch[3 * _EMBED_DIM:4 * _EMBED_DIM]
    prow = scratch[4 * _EMBED_DIM:5 * _EMBED_DIM]
    nrow = scratch[5 * _EMBED_DIM:6 * _EMBED_DIM]
    (qidx_v, alpha_v, diff_v, ssq_v,
     sem_s, sem_u, sem_p, sem_n, sem_a) = scratch[6 * _EMBED_DIM:]

    wid = lax.axis_index("s") * _NC + lax.axis_index("c")
    base = wid * _BPW

    pltpu.async_copy(neg_hbm.at[pl.ds(base, _BPW)], qidx_v, sem_s)
    for d in range(_EMBED_DIM):
        pltpu.async_copy(uoff_hbm.at[d, pl.ds(base, _BPW)], uidx[d], sem_s)
        pltpu.async_copy(poff_hbm.at[d, pl.ds(base, _BPW)], pidx[d], sem_s)
        pltpu.async_copy(noff_hbm.at[d, pl.ds(base, _BPW)], nidx[d], sem_s)
    for d in range(_EMBED_DIM):
        pltpu.make_async_copy(neg_hbm.at[pl.ds(0, _BPW)], uidx[d],
                              sem_s).wait()
        pltpu.make_async_copy(neg_hbm.at[pl.ds(0, _BPW)], pidx[d],
                              sem_s).wait()
        pltpu.make_async_copy(neg_hbm.at[pl.ds(0, _BPW)], nidx[d],
                              sem_s).wait()
    pltpu.make_async_copy(neg_hbm.at[pl.ds(0, _BPW)], qidx_v, sem_s).wait()

    for d in range(_EMBED_DIM):
        pltpu.async_copy(uflat_hbm.at[uidx[d]], urow[d], sem_u)
        pltpu.async_copy(iflat_hbm.at[pidx[d]], prow[d], sem_p)
        pltpu.async_copy(iflat_hbm.at[nidx[d]], nrow[d], sem_n)

    def issue_alpha(g, carry):
        iq_vec = qidx_v[pl.ds(g * 16, 16)]
        pltpu.async_copy(pop_hbm.at[iq_vec],
                         alpha_v.at[g // 8, pl.ds((g % 8) * 16, 16)], sem_a)
        return carry

    lax.fori_loop(0, _G, issue_alpha, 0)

    for d in range(_EMBED_DIM):
        pltpu.make_async_copy(neg_hbm.at[pl.ds(0, _BPW)], urow[d],
                              sem_u).wait()
        pltpu.make_async_copy(neg_hbm.at[pl.ds(0, _BPW)], prow[d],
                              sem_p).wait()
        pltpu.make_async_copy(neg_hbm.at[pl.ds(0, _BPW)], nrow[d],
                              sem_n).wait()
    pltpu.make_async_copy(alpha_out.at[pl.ds(0, 4)], alpha_v, sem_a).wait()

    def compute(g, ssq):
        sl = pl.ds(g * 16, 16)
        pos_acc = jnp.zeros((16,), jnp.float32)
        neg_acc = jnp.zeros((16,), jnp.float32)
        for d in range(_EMBED_DIM):
            u = urow[d][sl]
            p = prow[d][sl]
            q = nrow[d][sl]
            pos_acc = pos_acc + u * p
            neg_acc = neg_acc + u * q
            ssq = ssq + u * u + p * p + q * q
        diff_v[sl] = neg_acc - pos_acc
        return ssq

    ssq = lax.fori_loop(0, _G, compute, jnp.zeros((16,), jnp.float32))
    for r in range(16):
        ssq_v[pl.ds(r * 16, 16)] = ssq if r == 0 else jnp.zeros(
            (16,), jnp.float32)

    pltpu.sync_copy(diff_v, diff_out.at[pl.ds(base, _BPW)])
    pltpu.sync_copy(alpha_v, alpha_out.at[pl.ds(wid * 4, 4)])
    pltpu.sync_copy(ssq_v, ssq_out.at[wid])


_sc_gather = pl.kernel(
    _sc_body,
    out_type=[
        jax.ShapeDtypeStruct((_BATCH,), jnp.float32),
        jax.ShapeDtypeStruct((_BATCH // 128, 128), jnp.float32),
        jax.ShapeDtypeStruct((_NW, 256), jnp.float32),
    ],
    mesh=plsc.VectorSubcoreMesh(core_axis_name="c", subcore_axis_name="s"),
    scratch_types=(
        [pltpu.VMEM((_BPW,), jnp.int32)] * (3 * _EMBED_DIM)
        + [pltpu.VMEM((_BPW,), jnp.float32)] * (3 * _EMBED_DIM)
        + [
            pltpu.VMEM((_BPW,), jnp.int32),
            pltpu.VMEM((4, 128), jnp.float32),
            pltpu.VMEM((_BPW,), jnp.float32),
            pltpu.VMEM((256,), jnp.float32),
            pltpu.SemaphoreType.DMA,
            pltpu.SemaphoreType.DMA,
            pltpu.SemaphoreType.DMA,
            pltpu.SemaphoreType.DMA,
            pltpu.SemaphoreType.DMA,
        ]
    ),
)


def _loss_body(diff_ref, alpha_ref, ssq_ref, out_ref):
    x = diff_ref[...]
    a = alpha_ref[...]
    sp = jnp.maximum(x, 0.0) + jnp.log(1.0 + jnp.exp(-jnp.abs(x)))
    wb = jnp.sum(a * sp)
    lane = lax.broadcasted_iota(jnp.int32, (_NW, 256), 1)
    ssq = jnp.sum(jnp.where(lane < 16, ssq_ref[...], 0.0))
    out_ref[0, 0] = wb * _BPR_SCALE + ssq * _REG_SCALE


_loss_call = pl.pallas_call(
    _loss_body,
    out_shape=jax.ShapeDtypeStruct((1, 1), jnp.float32),
    in_specs=[
        pl.BlockSpec(memory_space=pltpu.VMEM),
        pl.BlockSpec(memory_space=pltpu.VMEM),
        pl.BlockSpec(memory_space=pltpu.VMEM),
    ],
    out_specs=pl.BlockSpec(memory_space=pltpu.SMEM),
)


def kernel(users, positive_items, negative_items, user_embedding,
           item_embedding, popularity):
    uflat = user_embedding.T.reshape(-1)
    iflat = item_embedding.T.reshape(-1)
    dvec = (jnp.arange(_EMBED_DIM, dtype=jnp.int32) * _NUM_USERS)[:, None]
    uoff = users[None, :] + dvec
    poff = positive_items[None, :] + dvec
    noff = negative_items[None, :] + dvec
    diff, alpha, ssq = _sc_gather(uoff, poff, noff, negative_items,
                                  uflat, iflat, popularity)
    loss = _loss_call(diff.reshape(128, 128), alpha, ssq)
    return loss[0, 0]


# speedup vs baseline: 8.5378x; 8.5280x over previous
import jax
import jax.numpy as jnp
from jax import lax
from jax.experimental import pallas as pl
from jax.experimental.pallas import tpu as pltpu
from jax.experimental.pallas import tpu_sc as plsc

_NUM_USERS = 1000000
_NUM_ITEMS = 1000000
_EMBED_DIM = 32
_TRAIN_INTERACTION_SIZE = 100000000
_WEIGHT_DECAY = 1e-4
_BATCH = 16384

_NC = 2
_NS = 16
_NW = _NC * _NS
_BPW = _BATCH // _NW
_RPW = _BPW * _EMBED_DIM // 128

_SPARSITY = _TRAIN_INTERACTION_SIZE / (_NUM_USERS * _NUM_ITEMS)
_BPR_SCALE = 1.0 / (_NUM_USERS * _SPARSITY * _BATCH)
_REG_SCALE = _WEIGHT_DECAY * 0.5 / _BATCH


def _sc_body(users_hbm, pos_hbm, neg_hbm, uemb_hbm, iemb_hbm, pop_hbm,
             urows_out, prows_out, nrows_out, alpha_out,
             uidx_v, pidx_v, nidx_v, urows_v, prows_v, nrows_v, alpha_v,
             sem_u, sem_p, sem_n, sem_a):
    wid = lax.axis_index("s") * _NC + lax.axis_index("c")
    base = wid * _BPW

    pltpu.sync_copy(users_hbm.at[pl.ds(base, _BPW)], uidx_v)
    pltpu.sync_copy(pos_hbm.at[pl.ds(base, _BPW)], pidx_v)
    pltpu.sync_copy(neg_hbm.at[pl.ds(base, _BPW)], nidx_v)

    def issue(g, carry):
        iu_vec = uidx_v[pl.ds(g * 16, 16)]
        ip_vec = pidx_v[pl.ds(g * 16, 16)]
        iq_vec = nidx_v[pl.ds(g * 16, 16)]
        pltpu.async_copy(pop_hbm.at[iq_vec],
                         alpha_v.at[g // 8, pl.ds((g % 8) * 16, 16)], sem_a)
        for k in range(16):
            b = g * 16 + k
            iu = iu_vec[k]
            ip = ip_vec[k]
            iq = iq_vec[k]
            r = b // 4
            c = (b % 4) * _EMBED_DIM
            pltpu.async_copy(uemb_hbm.at[iu],
                             urows_v.at[r, pl.ds(c, _EMBED_DIM)], sem_u)
            pltpu.async_copy(iemb_hbm.at[ip],
                             prows_v.at[r, pl.ds(c, _EMBED_DIM)], sem_p)
            pltpu.async_copy(iemb_hbm.at[iq],
                             nrows_v.at[r, pl.ds(c, _EMBED_DIM)], sem_n)
        return carry

    lax.fori_loop(0, _BPW // 16, issue, 0)

    pltpu.make_async_copy(urows_out.at[pl.ds(0, _RPW), :], urows_v,
                          sem_u).wait()
    pltpu.make_async_copy(prows_out.at[pl.ds(0, _RPW), :], prows_v,
                          sem_p).wait()
    pltpu.make_async_copy(nrows_out.at[pl.ds(0, _RPW), :], nrows_v,
                          sem_n).wait()
    pltpu.make_async_copy(urows_out.at[pl.ds(0, _BPW // 128), :], alpha_v,
                          sem_a).wait()

    out_base = wid * _RPW
    pltpu.sync_copy(urows_v, urows_out.at[pl.ds(out_base, _RPW)])
    pltpu.sync_copy(prows_v, prows_out.at[pl.ds(out_base, _RPW)])
    pltpu.sync_copy(nrows_v, nrows_out.at[pl.ds(out_base, _RPW)])
    pltpu.sync_copy(alpha_v, alpha_out.at[pl.ds(wid * (_BPW // 128),
                                                _BPW // 128)])


_sc_gather = pl.kernel(
    _sc_body,
    out_type=[
        jax.ShapeDtypeStruct((_NW * _RPW, 128), jnp.float32),
        jax.ShapeDtypeStruct((_NW * _RPW, 128), jnp.float32),
        jax.ShapeDtypeStruct((_NW * _RPW, 128), jnp.float32),
        jax.ShapeDtypeStruct((_BATCH // 128, 128), jnp.float32),
    ],
    mesh=plsc.VectorSubcoreMesh(core_axis_name="c", subcore_axis_name="s"),
    scratch_types=[
        pltpu.VMEM((_BPW,), jnp.int32),
        pltpu.VMEM((_BPW,), jnp.int32),
        pltpu.VMEM((_BPW,), jnp.int32),
        pltpu.VMEM((_RPW, 128), jnp.float32),
        pltpu.VMEM((_RPW, 128), jnp.float32),
        pltpu.VMEM((_RPW, 128), jnp.float32),
        pltpu.VMEM((_BPW // 128, 128), jnp.float32),
        pltpu.SemaphoreType.DMA,
        pltpu.SemaphoreType.DMA,
        pltpu.SemaphoreType.DMA,
        pltpu.SemaphoreType.DMA,
    ],
)


def _loss_body(u_ref, p_ref, n_ref, a_ref, out_ref):
    u = u_ref[...]
    p = p_ref[...]
    q = n_ref[...]
    x = u * (q - p)
    rows = lax.broadcasted_iota(jnp.int32, (128, 4), 0) // _EMBED_DIM
    cols = lax.broadcasted_iota(jnp.int32, (128, 4), 1)
    seg = (rows == cols).astype(jnp.float32)
    d4 = jnp.dot(x, seg, preferred_element_type=jnp.float32)
    sp = jnp.maximum(d4, 0.0) + jnp.log(1.0 + jnp.exp(-jnp.abs(d4)))
    wb = jnp.sum(a_ref[...] * sp)
    ssq = jnp.sum(u * u) + jnp.sum(p * p) + jnp.sum(q * q)
    out_ref[0, 0] = wb * _BPR_SCALE + ssq * _REG_SCALE


_loss_call = pl.pallas_call(
    _loss_body,
    out_shape=jax.ShapeDtypeStruct((1, 1), jnp.float32),
    in_specs=[
        pl.BlockSpec(memory_space=pltpu.VMEM),
        pl.BlockSpec(memory_space=pltpu.VMEM),
        pl.BlockSpec(memory_space=pltpu.VMEM),
        pl.BlockSpec(memory_space=pltpu.VMEM),
    ],
    out_specs=pl.BlockSpec(memory_space=pltpu.SMEM),
)


def kernel(users, positive_items, negative_items, user_embedding,
           item_embedding, popularity):
    urows, prows, nrows, alpha = _sc_gather(
        users, positive_items, negative_items,
        user_embedding, item_embedding, popularity)
    alpha4 = alpha.reshape(_BATCH).reshape(_BATCH // 4, 4)
    loss = _loss_call(urows, prows, nrows, alpha4)
    return loss[0, 0]
